# baseline (speedup 1.0000x reference)
import functools

import jax
import jax.numpy as jnp
from jax import lax
from jax.experimental import pallas as pl
from jax.experimental.pallas import tpu as pltpu
from jax.experimental.pallas import tpu_sc as plsc


def _emb_lookup(idx, table4, n_rows, d, n_workers, chunk):
    rows_per_w = n_rows // n_workers
    n_chunks = rows_per_w // chunk
    assert rows_per_w % chunk == 0 and chunk % 16 == 0
    mesh = plsc.VectorSubcoreMesh(core_axis_name="c", subcore_axis_name="s")

    @functools.partial(
        pl.kernel,
        mesh=mesh,
        out_type=jax.ShapeDtypeStruct((n_rows, d), jnp.float32),
        scratch_types=[
            pltpu.VMEM((chunk,), jnp.int32),
            pltpu.VMEM((chunk,), jnp.int32),
            pltpu.VMEM((chunk, 4 * d), jnp.float32),
            pltpu.VMEM((chunk, d), jnp.float32),
            pltpu.SemaphoreType.DMA,
        ],
        compiler_params=pltpu.CompilerParams(needs_layout_passes=False),
    )
    def emb_kernel(idx_hbm, table_hbm, out_hbm, xi, qv, rb, ob, sem):
        wid = lax.axis_index("s") * 2 + lax.axis_index("c")
        base = wid * rows_per_w
        n_groups = chunk // 16

        def body(i, carry):
            off = base + i * chunk
            pltpu.sync_copy(idx_hbm.at[pl.ds(off, chunk)], xi)

            def qbody(k, c):
                s = k * 16
                qv[pl.ds(s, 16)] = lax.shift_right_logical(xi[pl.ds(s, 16)], 2)
                return c

            lax.fori_loop(0, n_groups, qbody, 0)
            pltpu.async_copy(table_hbm.at[qv], rb, sem).wait()

            def cbody(k, c):
                s = k * 16
                row16 = lax.iota(jnp.int32, 16) + s
                colb = lax.shift_left(
                    lax.bitwise_and(xi[pl.ds(s, 16)], jnp.int32(3)), 5
                )
                for j in range(d):
                    vals = plsc.load_gather(rb, [row16, colb + j])
                    plsc.store_scatter(
                        ob, [row16, jnp.full((16,), j, jnp.int32)], vals
                    )
                return c

            lax.fori_loop(0, n_groups, cbody, 0)
            pltpu.sync_copy(ob, out_hbm.at[pl.ds(off, chunk)])
            return carry

        lax.fori_loop(0, n_chunks, body, 0)

    return emb_kernel(idx, table4)


def kernel(x, embd):
    b, f = x.shape
    v, d = embd.shape
    n_rows = b * f
    table4 = embd.reshape(v // 4, 4 * d)
    out = _emb_lookup(x.reshape(n_rows), table4, n_rows, d, 32, 256)
    return out.reshape(b, f, d)


# speedup vs baseline: 1.8284x; 1.8284x over previous
import functools

import jax
import jax.numpy as jnp
from jax import lax
from jax.experimental import pallas as pl
from jax.experimental.pallas import tpu as pltpu
from jax.experimental.pallas import tpu_sc as plsc


def _emb_lookup(idx2, embd, bsz, fld, d):
    tbs = bsz // 128
    n_chunks = fld * tbs
    n_workers = 32
    per_w = n_chunks // n_workers
    nbuf = 4
    m_rows = fld * (d // 8) * tbs * 8
    mesh = plsc.VectorSubcoreMesh(core_axis_name="c", subcore_axis_name="s")

    scratch = (
        [pltpu.VMEM((128,), jnp.int32) for _ in range(nbuf)]
        + [pltpu.VMEM((128, d), jnp.float32) for _ in range(nbuf)]
        + [pltpu.VMEM((d, 128), jnp.float32) for _ in range(nbuf)]
        + [pltpu.SemaphoreType.DMA for _ in range(3 * nbuf)]
    )

    @functools.partial(
        pl.kernel,
        mesh=mesh,
        out_type=jax.ShapeDtypeStruct((m_rows, 128), jnp.float32),
        scratch_types=scratch,
        compiler_params=pltpu.CompilerParams(
            use_tc_tiling_on_sc=False, needs_layout_passes=False
        ),
    )
    def emb_kernel(idx_hbm, table_hbm, out2_hbm, *bufs):
        xi = bufs[:nbuf]
        gb = bufs[nbuf : 2 * nbuf]
        segb = bufs[2 * nbuf : 3 * nbuf]
        si = bufs[3 * nbuf : 4 * nbuf]
        sg = bufs[4 * nbuf : 5 * nbuf]
        so = bufs[5 * nbuf :]
        wid = lax.axis_index("s") * 2 + lax.axis_index("c")
        c0 = wid * per_w

        def idx_off(k):
            c = c0 + k
            f = lax.shift_right_logical(c, 7)
            tb = lax.bitwise_and(c, jnp.int32(127))
            return f * bsz + tb * 128

        def out_rows(k):
            c = c0 + k
            f = lax.shift_right_logical(c, 7)
            tb = lax.bitwise_and(c, jnp.int32(127))
            return ((f * (d // 8)) * tbs + tb) * 8

        def stage_idx(k, b):
            return pltpu.async_copy(idx_hbm.at[pl.ds(idx_off(k), 128)], xi[b], si[b])

        def start_gather(k, b):
            return pltpu.async_copy(table_hbm.at[xi[b]], gb[b], sg[b])

        def transpose(b):
            def tbody(g, carry):
                r16 = lax.iota(jnp.int32, 16) + g * 16
                for j in range(d):
                    vals = plsc.load_gather(gb[b], [r16, jnp.full((16,), j, jnp.int32)])
                    segb[b][j, pl.ds(g * 16, 16)] = vals
                return carry

            lax.fori_loop(0, 128 // 16, tbody, 0)

        def start_out(k, b):
            m0 = out_rows(k)
            cps = []
            for tj in range(d // 8):
                cps.append(
                    pltpu.async_copy(
                        segb[b].at[pl.ds(tj * 8, 8)],
                        out2_hbm.at[pl.ds(m0 + tj * tbs * 8, 8)],
                        so[b],
                    )
                )
            return cps

        def drain_out(k, b):
            m0 = out_rows(k)
            for tj in range(d // 8):
                pltpu.make_async_copy(
                    segb[b].at[pl.ds(tj * 8, 8)],
                    out2_hbm.at[pl.ds(m0 + tj * tbs * 8, 8)],
                    so[b],
                ).wait()

        for b in range(nbuf):
            stage_idx(b, b)
        pltpu.make_async_copy(
            idx_hbm.at[pl.ds(idx_off(0), 128)], xi[0], si[0]
        ).wait()
        start_gather(0, 0)

        n_groups = per_w // nbuf

        def group(g, carry):
            for b in range(nbuf):
                k = g * nbuf + b
                bn = (b + 1) % nbuf
                @pl.when(k + 1 < per_w)
                def _():
                    pltpu.make_async_copy(
                        idx_hbm.at[pl.ds(idx_off(k + 1), 128)], xi[bn], si[bn]
                    ).wait()
                    start_gather(k + 1, bn)

                @pl.when(k >= nbuf)
                def _():
                    drain_out(k - nbuf, b)

                pltpu.make_async_copy(table_hbm.at[xi[b]], gb[b], sg[b]).wait()
                transpose(b)
                start_out(k, b)

                @pl.when(k + nbuf < per_w)
                def _():
                    stage_idx(k + nbuf, b)

            return carry

        lax.fori_loop(0, n_groups, group, 0)
        for b in range(nbuf):
            drain_out(per_w - nbuf + b, b)

    return emb_kernel(idx2, embd)


def kernel(x, embd):
    bsz, fld = x.shape
    v, d = embd.shape
    idx2 = x.T.reshape(bsz * fld)
    out2 = _emb_lookup(idx2, embd, bsz, fld, d)
    o = out2.reshape(fld, d // 8, bsz // 128, 8, 128)
    o = o.transpose(2, 4, 0, 1, 3)
    return o.reshape(bsz, fld, d)
